# baseline (speedup 1.0000x reference)
import math

import jax
import jax.numpy as jnp
from jax import lax
from jax.experimental import pallas as pl
from jax.experimental.pallas import tpu as pltpu
from jax.experimental.pallas import tpu_sc as plsc

D_VOCAB = 1000000
D_X = 32
D_P = 16
MAX_LEN = 2048
B = 1024
L = 50

NC, NS = 2, 16
NW = NC * NS
BPW = B // NW
TPW = BPW * L

_MS = 1.0 / math.sqrt(math.sqrt(2.0) - 1.0)
_K1 = math.sqrt(D_X) * _MS


def _body(srcT_hbm, tok_hbm, pos_hbm, w_hbm, b_hbm, out_hbm,
          ids_v, rows_v, pos_v, w_v, b_v, q_v, r_v, buf0, buf1,
          gsem, sem0, sem1):
    wid = lax.axis_index("s") * NC + lax.axis_index("c")
    wbt = wid // 4
    wbo = (wid % 4) * BPW

    pltpu.sync_copy(srcT_hbm.at[:, pl.ds(wid * BPW, BPW)], ids_v)
    gathers = [
        pltpu.async_copy(tok_hbm.at[ids_v.at[l]],
                         rows_v.at[pl.ds(l * BPW, BPW)], gsem)
        for l in range(L)
    ]

    pltpu.sync_copy(pos_hbm.at[pl.ds(0, 56)], pos_v)
    pltpu.sync_copy(w_hbm, w_v)
    pltpu.sync_copy(b_hbm, b_v)
    wv = w_v[...]
    bv = b_v[...]

    def pos_loop(l, carry):
        lf = l.astype(jnp.float32) * (1.0 / MAX_LEN)
        p = (0.5 * _MS) * (pos_v[l, :] + lf * wv + bv) + 1.0
        q_v[l, :] = _K1 * p
        r_v[l, :] = p - 1.0
        return carry

    lax.fori_loop(0, L, pos_loop, 0)

    for g in gathers:
        g.wait()

    zc = jnp.zeros((16,), jnp.int32)
    iota = lax.iota(jnp.int32, 16)

    def do_pos(l, buf):
        zl = zc + l

        def iblk_loop(iblk, carry):
            for bblk in range(2):
                rowi = iota + (l * BPW + bblk * 16)
                gs = [
                    plsc.load_gather(rows_v, [rowi, zc + (iblk * 8 + k)])
                    for k in range(8)
                ]
                for j in range(D_P):
                    qs = plsc.load_gather(q_v, [zl, zc + j])
                    rs = plsc.load_gather(r_v, [zl, zc + j])
                    for k in range(8):
                        buf[iblk * 8 + k, j // 8, j % 8,
                            pl.ds(bblk * 16, 16)] = gs[k] * qs + rs
            return carry

        lax.fori_loop(0, 4, iblk_loop, 0)

    def out_slice(l):
        return out_hbm.at[l, :, :, wbt, :, pl.ds(wbo, BPW)]

    def sp_loop(sp, carry):
        l0 = 2 * sp
        l1 = l0 + 1

        @pl.when(sp >= 1)
        def _w0():
            pltpu.make_async_copy(buf0, out_hbm.at[0, :, :, 0, :, pl.ds(0, BPW)],
                                  sem0).wait()

        do_pos(l0, buf0)
        pltpu.async_copy(buf0, out_slice(l0), sem0)

        @pl.when(sp >= 1)
        def _w1():
            pltpu.make_async_copy(buf1, out_hbm.at[0, :, :, 0, :, pl.ds(0, BPW)],
                                  sem1).wait()

        do_pos(l1, buf1)
        pltpu.async_copy(buf1, out_slice(l1), sem1)
        return carry

    lax.fori_loop(0, L // 2, sp_loop, 0)
    pltpu.make_async_copy(buf0, out_hbm.at[0, :, :, 0, :, pl.ds(0, BPW)],
                          sem0).wait()
    pltpu.make_async_copy(buf1, out_hbm.at[0, :, :, 0, :, pl.ds(0, BPW)],
                          sem1).wait()


def kernel(src, tok_table, pos_table, W, b):
    srcT = src.T
    wf = W.reshape(D_P)
    mesh = plsc.VectorSubcoreMesh(core_axis_name="c", subcore_axis_name="s")
    run = pl.kernel(
        _body,
        out_type=jax.ShapeDtypeStruct((L, D_X, 2, 8, 8, 128), jnp.float32),
        mesh=mesh,
        compiler_params=pltpu.CompilerParams(
            needs_layout_passes=False, use_tc_tiling_on_sc=False),
        scratch_types=[
            pltpu.VMEM((L, BPW), jnp.int32),
            pltpu.VMEM((TPW, D_X), jnp.float32),
            pltpu.VMEM((56, D_P), jnp.float32),
            pltpu.VMEM((D_P,), jnp.float32),
            pltpu.VMEM((D_P,), jnp.float32),
            pltpu.VMEM((L, D_P), jnp.float32),
            pltpu.VMEM((L, D_P), jnp.float32),
            pltpu.VMEM((D_X, 2, 8, BPW), jnp.float32),
            pltpu.VMEM((D_X, 2, 8, BPW), jnp.float32),
            pltpu.SemaphoreType.DMA,
            pltpu.SemaphoreType.DMA,
            pltpu.SemaphoreType.DMA,
        ],
    )
    out6 = run(srcT, tok_table, pos_table, wf, b)
    return out6.transpose(3, 5, 0, 1, 2, 4).reshape(B, L, D_X, D_P)


# speedup vs baseline: 1.0059x; 1.0059x over previous
import math

import jax
import jax.numpy as jnp
from jax import lax
from jax.experimental import pallas as pl
from jax.experimental.pallas import tpu as pltpu
from jax.experimental.pallas import tpu_sc as plsc

D_VOCAB = 1000000
D_X = 32
D_P = 16
MAX_LEN = 2048
B = 1024
L = 50

NC, NS = 2, 16
NW = NC * NS
BPW = B // NW
ROWS4 = D_VOCAB // 4

_MS = 1.0 / math.sqrt(math.sqrt(2.0) - 1.0)
_K1 = math.sqrt(D_X) * _MS


def _body(srcT_hbm, tok_hbm, pos_hbm, w_hbm, b_hbm, out_hbm,
          ids_v, idq_v, colb_v, rbuf0, rbuf1, pos_v, w_v, b_v, q_v, r_v,
          buf0, buf1, gsem, sem0, sem1):
    wid = lax.axis_index("s") * NC + lax.axis_index("c")
    wbt = wid // 4
    wbo = (wid % 4) * BPW

    pltpu.sync_copy(srcT_hbm.at[:, pl.ds(wid * BPW, BPW)], ids_v)

    def idx_loop(l, carry):
        for h in range(2):
            v = ids_v[l, pl.ds(h * 16, 16)]
            idq_v[l, pl.ds(h * 16, 16)] = lax.shift_right_logical(v, 2)
            colb_v[l, pl.ds(h * 16, 16)] = (v & 3) * D_X
        return carry

    lax.fori_loop(0, L, idx_loop, 0)

    pltpu.sync_copy(pos_hbm.at[pl.ds(0, 56)], pos_v)
    pltpu.sync_copy(w_hbm, w_v)
    pltpu.sync_copy(b_hbm, b_v)
    wv = w_v[...]
    bv = b_v[...]

    def pos_loop(l, carry):
        lf = l.astype(jnp.float32) * (1.0 / MAX_LEN)
        p = (0.5 * _MS) * (pos_v[l, :] + lf * wv + bv) + 1.0
        q_v[l, :] = _K1 * p
        r_v[l, :] = p - 1.0
        return carry

    lax.fori_loop(0, L, pos_loop, 0)

    zc = jnp.zeros((16,), jnp.int32)
    iota = lax.iota(jnp.int32, 16)

    def fire_gather(l, rbuf):
        pltpu.async_copy(tok_hbm.at[idq_v.at[l]], rbuf, gsem)

    def drain_gather(rbuf):
        pltpu.make_async_copy(tok_hbm.at[pl.ds(0, BPW)], rbuf, gsem).wait()

    def do_pos(l, rbuf, buf):
        zl = zc + l
        qs = [plsc.load_gather(q_v, [zl, zc + j]) for j in range(D_P)]
        rs = [plsc.load_gather(r_v, [zl, zc + j]) for j in range(D_P)]

        for bblk in range(2):
            rowi = iota + (bblk * 16)
            colb = colb_v[l, pl.ds(bblk * 16, 16)]

            def iblk_loop(iblk, carry):
                gs = [
                    plsc.load_gather(rbuf, [rowi, colb + (iblk * 8 + k)])
                    for k in range(8)
                ]
                for j in range(D_P):
                    for k in range(8):
                        buf[iblk * 8 + k, j // 8, j % 8,
                            pl.ds(bblk * 16, 16)] = gs[k] * qs[j] + rs[j]
                return carry

            lax.fori_loop(0, 4, iblk_loop, 0)

    def out_slice(l):
        return out_hbm.at[l, :, :, wbt, :, pl.ds(wbo, BPW)]

    fire_gather(0, rbuf0)

    def sp_loop(sp, carry):
        l0 = 2 * sp
        l1 = l0 + 1

        drain_gather(rbuf0)
        fire_gather(l1, rbuf1)

        @pl.when(sp >= 1)
        def _w0():
            pltpu.make_async_copy(buf0, out_hbm.at[0, :, :, 0, :, pl.ds(0, BPW)],
                                  sem0).wait()

        do_pos(l0, rbuf0, buf0)
        pltpu.async_copy(buf0, out_slice(l0), sem0)

        drain_gather(rbuf1)

        @pl.when(l1 + 1 < L)
        def _g():
            fire_gather(l1 + 1, rbuf0)

        @pl.when(sp >= 1)
        def _w1():
            pltpu.make_async_copy(buf1, out_hbm.at[0, :, :, 0, :, pl.ds(0, BPW)],
                                  sem1).wait()

        do_pos(l1, rbuf1, buf1)
        pltpu.async_copy(buf1, out_slice(l1), sem1)
        return carry

    lax.fori_loop(0, L // 2, sp_loop, 0)
    pltpu.make_async_copy(buf0, out_hbm.at[0, :, :, 0, :, pl.ds(0, BPW)],
                          sem0).wait()
    pltpu.make_async_copy(buf1, out_hbm.at[0, :, :, 0, :, pl.ds(0, BPW)],
                          sem1).wait()


def kernel(src, tok_table, pos_table, W, b):
    srcT = src.T
    wf = W.reshape(D_P)
    tokT = lax.optimization_barrier(tok_table.T)
    tok4 = tokT.T.reshape(ROWS4, 4 * D_X)
    mesh = plsc.VectorSubcoreMesh(core_axis_name="c", subcore_axis_name="s")
    run = pl.kernel(
        _body,
        out_type=jax.ShapeDtypeStruct((L, D_X, 2, 8, 8, 128), jnp.float32),
        mesh=mesh,
        compiler_params=pltpu.CompilerParams(
            needs_layout_passes=False, use_tc_tiling_on_sc=False),
        scratch_types=[
            pltpu.VMEM((L, BPW), jnp.int32),
            pltpu.VMEM((L, BPW), jnp.int32),
            pltpu.VMEM((L, BPW), jnp.int32),
            pltpu.VMEM((BPW, 4 * D_X), jnp.float32),
            pltpu.VMEM((BPW, 4 * D_X), jnp.float32),
            pltpu.VMEM((56, D_P), jnp.float32),
            pltpu.VMEM((D_P,), jnp.float32),
            pltpu.VMEM((D_P,), jnp.float32),
            pltpu.VMEM((L, D_P), jnp.float32),
            pltpu.VMEM((L, D_P), jnp.float32),
            pltpu.VMEM((D_X, 2, 8, BPW), jnp.float32),
            pltpu.VMEM((D_X, 2, 8, BPW), jnp.float32),
            pltpu.SemaphoreType.DMA,
            pltpu.SemaphoreType.DMA,
            pltpu.SemaphoreType.DMA,
        ],
    )
    out6 = run(srcT, tok4, pos_table, wf, b)
    return out6.transpose(3, 5, 0, 1, 2, 4).reshape(B, L, D_X, D_P)


# speedup vs baseline: 1.5701x; 1.5608x over previous
import math

import jax
import jax.numpy as jnp
from jax import lax
from jax.experimental import pallas as pl
from jax.experimental.pallas import tpu as pltpu
from jax.experimental.pallas import tpu_sc as plsc

D_VOCAB = 1000000
D_X = 32
D_P = 16
MAX_LEN = 2048
B = 1024
L = 50

NC, NS = 2, 16
NW = NC * NS
BPW = B // NW
TPW = BPW * L
TROWS = (B * L) // 4
TRPW = TROWS // NW

_MS = 1.0 / math.sqrt(math.sqrt(2.0) - 1.0)
_K1 = math.sqrt(D_X) * _MS


def _g_body(srcw_hbm, tok_hbm, temb_hbm, ids_v, rbuf0, rbuf1, stage_v, gsem):
    wid = lax.axis_index("s") * NC + lax.axis_index("c")
    pltpu.sync_copy(srcw_hbm.at[wid], ids_v)

    zc = jnp.zeros((16,), jnp.int32)
    iota = lax.iota(jnp.int32, 16)

    def fire(l, rbuf):
        for h in range(2):
            vv = ids_v[l, pl.ds(h * 16, 16)]
            for t in range(16):
                pltpu.async_copy(tok_hbm.at[lax.shift_right_logical(vv[t], 3)],
                                 rbuf.at[h * 16 + t], gsem)

    def drain(rbuf):
        for t in range(BPW):
            pltpu.make_async_copy(tok_hbm.at[0], rbuf.at[t], gsem).wait()

    def extract(l, rbuf):
        for h in range(2):
            vv = ids_v[l, pl.ds(h * 16, 16)]
            for t in range(16):
                tt = h * 16 + t
                vr = zc + (vv[t] & 7)
                for k in range(2):
                    g = plsc.load_gather(rbuf, [zc + tt, vr, iota + k * 16])
                    stage_v[(l * BPW + tt) // 4,
                            pl.ds((tt & 3) * D_X + k * 16, 16)] = g

    fire(0, rbuf0)

    def sp_loop(sp, carry):
        l0 = 2 * sp
        l1 = l0 + 1
        drain(rbuf0)
        fire(l1, rbuf1)
        extract(l0, rbuf0)
        drain(rbuf1)

        @pl.when(l1 + 1 < L)
        def _f():
            fire(l1 + 1, rbuf0)

        extract(l1, rbuf1)
        return carry

    lax.fori_loop(0, L // 2, sp_loop, 0)
    pltpu.sync_copy(stage_v, temb_hbm.at[pl.ds(wid * TRPW, TRPW)])


def _e_body(temb_hbm, pos_hbm, w_hbm, b_hbm, out_hbm,
            tslab_v, pos_v, w_v, b_v, q_v, r_v, buf0, buf1,
            tsem, sem0, sem1):
    wid = lax.axis_index("s") * NC + lax.axis_index("c")
    wbt = wid // 4
    wbo = (wid % 4) * BPW

    cp = pltpu.async_copy(temb_hbm.at[pl.ds(wid * TRPW, TRPW)], tslab_v, tsem)

    pltpu.sync_copy(pos_hbm.at[pl.ds(0, 56)], pos_v)
    pltpu.sync_copy(w_hbm, w_v)
    pltpu.sync_copy(b_hbm, b_v)
    wv = w_v[...]
    bv = b_v[...]

    def pos_loop(l, carry):
        lf = l.astype(jnp.float32) * (1.0 / MAX_LEN)
        p = (0.5 * _MS) * (pos_v[l, :] + lf * wv + bv) + 1.0
        q_v[l, :] = _K1 * p
        r_v[l, :] = p - 1.0
        return carry

    lax.fori_loop(0, L, pos_loop, 0)
    cp.wait()

    zc = jnp.zeros((16,), jnp.int32)
    iota = lax.iota(jnp.int32, 16)
    iota_d4 = lax.shift_right_logical(iota, 2)
    colpat = (iota & 3) * D_X

    def do_pos(l, buf):
        zl = zc + l
        qs = [plsc.load_gather(q_v, [zl, zc + j]) for j in range(D_P)]
        rs = [plsc.load_gather(r_v, [zl, zc + j]) for j in range(D_P)]

        for bblk in range(2):
            rowi = iota_d4 + (l * 8 + bblk * 4)

            def iblk_loop(iblk, carry):
                gs = [
                    plsc.load_gather(tslab_v, [rowi, colpat + (iblk * 8 + k)])
                    for k in range(8)
                ]
                for j in range(D_P):
                    for k in range(8):
                        buf[iblk * 8 + k, j // 8, j % 8,
                            pl.ds(bblk * 16, 16)] = gs[k] * qs[j] + rs[j]
                return carry

            lax.fori_loop(0, 4, iblk_loop, 0)

    def out_slice(l):
        return out_hbm.at[l, :, :, wbt, :, pl.ds(wbo, BPW)]

    def sp_loop(sp, carry):
        l0 = 2 * sp
        l1 = l0 + 1

        @pl.when(sp >= 1)
        def _w0():
            pltpu.make_async_copy(buf0, out_hbm.at[0, :, :, 0, :, pl.ds(0, BPW)],
                                  sem0).wait()

        do_pos(l0, buf0)
        pltpu.async_copy(buf0, out_slice(l0), sem0)

        @pl.when(sp >= 1)
        def _w1():
            pltpu.make_async_copy(buf1, out_hbm.at[0, :, :, 0, :, pl.ds(0, BPW)],
                                  sem1).wait()

        do_pos(l1, buf1)
        pltpu.async_copy(buf1, out_slice(l1), sem1)
        return carry

    lax.fori_loop(0, L // 2, sp_loop, 0)
    pltpu.make_async_copy(buf0, out_hbm.at[0, :, :, 0, :, pl.ds(0, BPW)],
                          sem0).wait()
    pltpu.make_async_copy(buf1, out_hbm.at[0, :, :, 0, :, pl.ds(0, BPW)],
                          sem1).wait()


def kernel(src, tok_table, pos_table, W, b):
    srcw = src.T.reshape(L, NW, BPW).transpose(1, 0, 2)
    wf = W.reshape(D_P)
    mesh = plsc.VectorSubcoreMesh(core_axis_name="c", subcore_axis_name="s")

    gather = pl.kernel(
        _g_body,
        out_type=jax.ShapeDtypeStruct((TROWS, 4 * D_X), jnp.float32),
        mesh=mesh,
        compiler_params=pltpu.CompilerParams(
            needs_layout_passes=False, use_tc_tiling_on_sc=True),
        scratch_types=[
            pltpu.VMEM((L, BPW), jnp.int32),
            pltpu.VMEM((BPW, 8, D_X), jnp.float32),
            pltpu.VMEM((BPW, 8, D_X), jnp.float32),
            pltpu.VMEM((TRPW, 4 * D_X), jnp.float32),
            pltpu.SemaphoreType.DMA,
        ],
    )
    temb = gather(srcw, tok_table.reshape(D_VOCAB // 8, 8, D_X))

    expand = pl.kernel(
        _e_body,
        out_type=jax.ShapeDtypeStruct((L, D_X, 2, 8, 8, 128), jnp.float32),
        mesh=mesh,
        compiler_params=pltpu.CompilerParams(
            needs_layout_passes=False, use_tc_tiling_on_sc=False),
        scratch_types=[
            pltpu.VMEM((TRPW, 4 * D_X), jnp.float32),
            pltpu.VMEM((56, D_P), jnp.float32),
            pltpu.VMEM((D_P,), jnp.float32),
            pltpu.VMEM((D_P,), jnp.float32),
            pltpu.VMEM((L, D_P), jnp.float32),
            pltpu.VMEM((L, D_P), jnp.float32),
            pltpu.VMEM((D_X, 2, 8, BPW), jnp.float32),
            pltpu.VMEM((D_X, 2, 8, BPW), jnp.float32),
            pltpu.SemaphoreType.DMA,
            pltpu.SemaphoreType.DMA,
            pltpu.SemaphoreType.DMA,
        ],
    )
    out6 = expand(temb, pos_table, wf, b)
    return out6.transpose(3, 5, 0, 1, 2, 4).reshape(B, L, D_X, D_P)
